# baseline (speedup 1.0000x reference)
import jax
import jax.numpy as jnp
from jax import lax
from jax.experimental import pallas as pl
from jax.experimental.pallas import tpu as pltpu
from jax.experimental.pallas import tpu_sc as plsc

NC = 2
NS = 16
NW = NC * NS
L = 16
K = 128


def _sc_mesh():
    return plsc.VectorSubcoreMesh(core_axis_name="c", subcore_axis_name="s",
                                  num_cores=NC, num_subcores=NS)


def _build_sc_agg(NP, D, CPT):
    out_type = jax.ShapeDtypeStruct((NC, NP, D), jnp.float32)
    CG = 16
    G = CPT // CG
    scratch = [
        pltpu.VMEM((CG, K), jnp.int32),
        pltpu.VMEM((CG, K), jnp.int32),
        pltpu.VMEM((CG, K), jnp.int32),
        pltpu.VMEM((CG, K), jnp.int32),
        pltpu.VMEM((K, D), jnp.float32),
        pltpu.VMEM((K, D), jnp.float32),
        pltpu.VMEM_SHARED((NP, D), jnp.float32),
        pltpu.SemaphoreType.DMA,
        pltpu.SemaphoreType.DMA,
        pltpu.SemaphoreType.DMA,
        pltpu.SemaphoreType.DMA,
        pltpu.SemaphoreType.DMA,
    ]

    def body(h_hbm, srcm_hbm, dstm_hbm, agg_out,
             sv0, dv0, sv1, dv1, rows_a, rows_b, agg_sh,
             sem_i, sem_a, sem_b, sem_sa, sem_sb):
        cid = lax.axis_index("c")
        sid = lax.axis_index("s")
        wid = cid * NS + sid
        rpt = NP // NS
        base = sid * rpt
        ibufs = [(sv0, dv0), (sv1, dv1)]

        def stage_idx(g, sv, dv):
            off = wid * CPT + g * CG
            cs = pltpu.async_copy(srcm_hbm.at[pl.ds(off, CG)], sv, sem_i)
            cd = pltpu.async_copy(dstm_hbm.at[pl.ds(off, CG)], dv, sem_i)
            return cs, cd

        cp_s, cp_d = stage_idx(0, sv0, dv0)

        zero16 = jnp.zeros((L,), jnp.float32)

        def zrow(r, _):
            for i in range(D // L):
                rows_a[r, pl.ds(i * L, L)] = zero16
            return 0

        lax.fori_loop(0, K, zrow, 0)
        for t in range(rpt // K):
            pltpu.sync_copy(rows_a, agg_sh.at[pl.ds(base + t * K, K)])

        cp_s.wait()
        cp_d.wait()
        plsc.subcore_barrier()

        def gather(sv, j, buf, sem):
            return pltpu.async_copy(h_hbm.at[sv.at[j]], buf, sem)

        def wait_gather(sv, j, buf, sem):
            pltpu.make_async_copy(h_hbm.at[sv.at[j]], buf, sem).wait()

        def scatter(dv, j, buf, sem):
            pltpu.async_copy(buf, agg_sh.at[dv.at[j]], sem, add=True)

        def wait_scatter(dv, j, buf, sem):
            pltpu.make_async_copy(buf, agg_sh.at[dv.at[j]], sem).wait()

        gather(sv0, 0, rows_a, sem_a)
        for g in range(G):
            sv, dv = ibufs[g % 2]
            nsv, ndv = ibufs[(g + 1) % 2]
            if g + 1 < G:
                stage_idx(g + 1, nsv, ndv)

            def pair(p, _, sv=sv, dv=dv):
                j0 = 2 * p
                wait_gather(sv, j0, rows_a, sem_a)
                scatter(dv, j0, rows_a, sem_sa)
                gather(sv, j0 + 1, rows_b, sem_b)
                wait_gather(sv, j0 + 1, rows_b, sem_b)
                scatter(dv, j0 + 1, rows_b, sem_sb)
                wait_scatter(dv, j0, rows_a, sem_sa)
                gather(sv, j0 + 2, rows_a, sem_a)
                wait_scatter(dv, j0 + 1, rows_b, sem_sb)
                return 0

            lax.fori_loop(0, CG // 2 - 1, pair, 0)
            j0 = CG - 2
            wait_gather(sv, j0, rows_a, sem_a)
            scatter(dv, j0, rows_a, sem_sa)
            gather(sv, j0 + 1, rows_b, sem_b)
            if g + 1 < G:
                off = wid * CPT + (g + 1) * CG
                pltpu.make_async_copy(
                    srcm_hbm.at[pl.ds(off, CG)], nsv, sem_i).wait()
                pltpu.make_async_copy(
                    dstm_hbm.at[pl.ds(off, CG)], ndv, sem_i).wait()
            wait_gather(sv, j0 + 1, rows_b, sem_b)
            scatter(dv, j0 + 1, rows_b, sem_sb)
            wait_scatter(dv, j0, rows_a, sem_sa)
            if g + 1 < G:
                gather(nsv, 0, rows_a, sem_a)
            wait_scatter(dv, j0 + 1, rows_b, sem_sb)

        plsc.subcore_barrier()
        pltpu.sync_copy(agg_sh.at[pl.ds(base, rpt)],
                        agg_out.at[cid, pl.ds(base, rpt)])

    return pl.kernel(body, out_type=out_type, mesh=_sc_mesh(),
                     scratch_types=scratch)


def _build_sc_deg(NP, EPT16):
    out_type = jax.ShapeDtypeStruct((NW, NP), jnp.float32)
    GG = 128
    scratch = [
        pltpu.VMEM((GG, L), jnp.int32),
        pltpu.VMEM((NP,), jnp.float32),
        pltpu.SemaphoreType.DMA,
    ]

    def body(dst16_hbm, deg_out, dst_v, deg_v, sem):
        cid = lax.axis_index("c")
        sid = lax.axis_index("s")
        wid = cid * NS + sid

        zero16 = jnp.zeros((L,), jnp.float32)

        def zdeg(r, _):
            deg_v[pl.ds(r * L, L)] = zero16
            return 0

        lax.fori_loop(0, NP // L, zdeg, 0)

        def step(q, _):
            for u in range(4):
                d16 = dst_v[4 * q + u, pl.ds(0, L)]
                cnt, last = plsc.scan_count(d16)
                plsc.addupdate_scatter(deg_v, [d16], cnt.astype(jnp.float32),
                                       mask=last)
            return 0

        for g in range(EPT16 // GG):
            pltpu.async_copy(
                dst16_hbm.at[pl.ds(wid * EPT16 + g * GG, GG)], dst_v, sem
            ).wait()
            lax.fori_loop(0, GG // 4, step, 0)

        pltpu.sync_copy(deg_v, deg_out.at[wid])

    return pl.kernel(
        body, out_type=out_type, mesh=_sc_mesh(), scratch_types=scratch,
        compiler_params=pltpu.CompilerParams(needs_layout_passes=False))


def _build_tc_layer(NP, D, relu):
    RB = 1024
    grid = (NP // RB,)

    def body(x_ref, aggp_ref, degp_ref, ws_ref, wn_ref, b_ref, o_ref):
        agg = aggp_ref[0] + aggp_ref[1]
        deg = jnp.sum(jnp.transpose(degp_ref[...]), axis=1, keepdims=True)
        inv = 1.0 / jnp.maximum(deg, 1.0)
        mean = agg * inv
        out = (jnp.dot(x_ref[...], ws_ref[...], preferred_element_type=jnp.float32)
               + jnp.dot(mean, wn_ref[...], preferred_element_type=jnp.float32)
               + b_ref[...])
        if relu:
            out = jnp.maximum(out, 0.0)
        o_ref[...] = out

    return pl.pallas_call(
        body,
        grid=grid,
        in_specs=[
            pl.BlockSpec((RB, D), lambda i: (i, 0)),
            pl.BlockSpec((NC, RB, D), lambda i: (0, i, 0)),
            pl.BlockSpec((NW, RB), lambda i: (0, i)),
            pl.BlockSpec((D, D), lambda i: (0, 0)),
            pl.BlockSpec((D, D), lambda i: (0, 0)),
            pl.BlockSpec((1, D), lambda i: (0, 0)),
        ],
        out_specs=pl.BlockSpec((RB, D), lambda i: (i, 0)),
        out_shape=jax.ShapeDtypeStruct((NP, D), jnp.float32),
    )


def kernel(in_feat, edge_index, W_self1, W_neigh1, b1, W_self2, W_neigh2, b2):
    N, D = in_feat.shape
    E = edge_index.shape[1]
    NP = ((N + 1 + NS * K - 1) // (NS * K)) * (NS * K)
    CPT = (E + NW * K - 1) // (NW * K)
    CPT = ((CPT + 7) // 8) * 8
    EP = NW * CPT * K
    pad_e = EP - E

    src = edge_index[0].astype(jnp.int32)
    dst = edge_index[1].astype(jnp.int32)
    pad_ar = jnp.arange(pad_e, dtype=jnp.int32)
    src = jnp.concatenate([src, pad_ar % N])
    dst = jnp.concatenate([dst, N + pad_ar % (NP - N)])
    srcm = src.reshape(NW * CPT, K)
    dstm = dst.reshape(NW * CPT, K)
    dst16 = dst.reshape(EP // L, L)

    x = jnp.concatenate([in_feat, jnp.zeros((NP - N, D), jnp.float32)])

    sc_agg = _build_sc_agg(NP, D, CPT)
    sc_deg = _build_sc_deg(NP, EP // L // NW)
    tc1 = _build_tc_layer(NP, D, relu=True)
    tc2 = _build_tc_layer(NP, D, relu=False)

    aggp1 = sc_agg(x, srcm, dstm)
    degp = sc_deg(dst16)
    h1 = tc1(x, aggp1, degp, W_self1.T, W_neigh1.T, b1.reshape(1, D))
    aggp2 = sc_agg(h1, srcm, dstm)
    out = tc2(h1, aggp2, degp, W_self2.T, W_neigh2.T, b2.reshape(1, D))
    return out[:N]


# speedup vs baseline: 1.1142x; 1.1142x over previous
import jax
import jax.numpy as jnp
from jax import lax
from jax.experimental import pallas as pl
from jax.experimental.pallas import tpu as pltpu
from jax.experimental.pallas import tpu_sc as plsc

NC = 2
NS = 16
NW = NC * NS
L = 16
K = 128


def _sc_mesh():
    return plsc.VectorSubcoreMesh(core_axis_name="c", subcore_axis_name="s",
                                  num_cores=NC, num_subcores=NS)


def _build_sc_agg(NP, D, CPT):
    out_type = jax.ShapeDtypeStruct((NC, NP, D), jnp.float32)
    CG = 16
    G = CPT // CG
    scratch = [
        pltpu.VMEM((CG, K), jnp.int32),
        pltpu.VMEM((CG, K), jnp.int32),
        pltpu.VMEM((CG, K), jnp.int32),
        pltpu.VMEM((CG, K), jnp.int32),
        pltpu.VMEM((K, D), jnp.float32),
        pltpu.VMEM((K, D), jnp.float32),
        pltpu.VMEM_SHARED((NP, D), jnp.float32),
        pltpu.SemaphoreType.DMA,
        pltpu.SemaphoreType.DMA,
        pltpu.SemaphoreType.DMA,
        pltpu.SemaphoreType.DMA,
        pltpu.SemaphoreType.DMA,
    ]

    def body(h_hbm, srcm_hbm, dstm_hbm, agg_out,
             sv0, dv0, sv1, dv1, rows_a, rows_b, agg_sh,
             sem_i, sem_a, sem_b, sem_sa, sem_sb):
        cid = lax.axis_index("c")
        sid = lax.axis_index("s")
        wid = cid * NS + sid
        rpt = NP // NS
        base = sid * rpt
        ibufs = [(sv0, dv0), (sv1, dv1)]

        def stage_idx(g, sv, dv):
            off = wid * CPT + g * CG
            cs = pltpu.async_copy(srcm_hbm.at[pl.ds(off, CG)], sv, sem_i)
            cd = pltpu.async_copy(dstm_hbm.at[pl.ds(off, CG)], dv, sem_i)
            return cs, cd

        cp_s, cp_d = stage_idx(0, sv0, dv0)

        zero16 = jnp.zeros((L,), jnp.float32)

        def zrow(r, _):
            for i in range(D // L):
                rows_a[r, pl.ds(i * L, L)] = zero16
            return 0

        lax.fori_loop(0, K, zrow, 0)
        for t in range(rpt // K):
            pltpu.sync_copy(rows_a, agg_sh.at[pl.ds(base + t * K, K)])

        cp_s.wait()
        cp_d.wait()
        plsc.subcore_barrier()

        def gather(sv, j, buf, sem):
            return pltpu.async_copy(h_hbm.at[sv.at[j]], buf, sem)

        def wait_gather(sv, j, buf, sem):
            pltpu.make_async_copy(h_hbm.at[sv.at[j]], buf, sem).wait()

        def scatter(dv, j, buf):
            pltpu.sync_copy(buf, agg_sh.at[dv.at[j]], add=True)

        gather(sv0, 0, rows_a, sem_a)
        for g in range(G):
            sv, dv = ibufs[g % 2]
            nsv, ndv = ibufs[(g + 1) % 2]
            if g + 1 < G:
                stage_idx(g + 1, nsv, ndv)

            def pair(p, _, sv=sv, dv=dv):
                j0 = 2 * p
                wait_gather(sv, j0, rows_a, sem_a)
                gather(sv, j0 + 1, rows_b, sem_b)
                scatter(dv, j0, rows_a)
                gather(sv, j0 + 2, rows_a, sem_a)
                wait_gather(sv, j0 + 1, rows_b, sem_b)
                scatter(dv, j0 + 1, rows_b)
                return 0

            lax.fori_loop(0, CG // 2 - 1, pair, 0)
            j0 = CG - 2
            wait_gather(sv, j0, rows_a, sem_a)
            gather(sv, j0 + 1, rows_b, sem_b)
            scatter(dv, j0, rows_a)
            if g + 1 < G:
                off = wid * CPT + (g + 1) * CG
                pltpu.make_async_copy(
                    srcm_hbm.at[pl.ds(off, CG)], nsv, sem_i).wait()
                pltpu.make_async_copy(
                    dstm_hbm.at[pl.ds(off, CG)], ndv, sem_i).wait()
                gather(nsv, 0, rows_a, sem_a)
            wait_gather(sv, j0 + 1, rows_b, sem_b)
            scatter(dv, j0 + 1, rows_b)

        plsc.subcore_barrier()
        pltpu.sync_copy(agg_sh.at[pl.ds(base, rpt)],
                        agg_out.at[cid, pl.ds(base, rpt)])

    return pl.kernel(body, out_type=out_type, mesh=_sc_mesh(),
                     scratch_types=scratch)


def _build_sc_deg(NP, EPT16):
    out_type = jax.ShapeDtypeStruct((NW, NP), jnp.float32)
    GG = 128
    scratch = [
        pltpu.VMEM((GG, L), jnp.int32),
        pltpu.VMEM((NP,), jnp.float32),
        pltpu.SemaphoreType.DMA,
    ]

    def body(dst16_hbm, deg_out, dst_v, deg_v, sem):
        cid = lax.axis_index("c")
        sid = lax.axis_index("s")
        wid = cid * NS + sid

        zero16 = jnp.zeros((L,), jnp.float32)

        def zdeg(r, _):
            deg_v[pl.ds(r * L, L)] = zero16
            return 0

        lax.fori_loop(0, NP // L, zdeg, 0)

        def step(q, _):
            for u in range(4):
                d16 = dst_v[4 * q + u, pl.ds(0, L)]
                cnt, last = plsc.scan_count(d16)
                plsc.addupdate_scatter(deg_v, [d16], cnt.astype(jnp.float32),
                                       mask=last)
            return 0

        for g in range(EPT16 // GG):
            pltpu.async_copy(
                dst16_hbm.at[pl.ds(wid * EPT16 + g * GG, GG)], dst_v, sem
            ).wait()
            lax.fori_loop(0, GG // 4, step, 0)

        pltpu.sync_copy(deg_v, deg_out.at[wid])

    return pl.kernel(
        body, out_type=out_type, mesh=_sc_mesh(), scratch_types=scratch,
        compiler_params=pltpu.CompilerParams(needs_layout_passes=False))


def _build_tc_layer(NP, D, relu):
    RB = 1024
    grid = (NP // RB,)

    def body(x_ref, aggp_ref, degp_ref, ws_ref, wn_ref, b_ref, o_ref):
        agg = aggp_ref[0] + aggp_ref[1]
        deg = jnp.sum(jnp.transpose(degp_ref[...]), axis=1, keepdims=True)
        inv = 1.0 / jnp.maximum(deg, 1.0)
        mean = agg * inv
        out = (jnp.dot(x_ref[...], ws_ref[...], preferred_element_type=jnp.float32)
               + jnp.dot(mean, wn_ref[...], preferred_element_type=jnp.float32)
               + b_ref[...])
        if relu:
            out = jnp.maximum(out, 0.0)
        o_ref[...] = out

    return pl.pallas_call(
        body,
        grid=grid,
        in_specs=[
            pl.BlockSpec((RB, D), lambda i: (i, 0)),
            pl.BlockSpec((NC, RB, D), lambda i: (0, i, 0)),
            pl.BlockSpec((NW, RB), lambda i: (0, i)),
            pl.BlockSpec((D, D), lambda i: (0, 0)),
            pl.BlockSpec((D, D), lambda i: (0, 0)),
            pl.BlockSpec((1, D), lambda i: (0, 0)),
        ],
        out_specs=pl.BlockSpec((RB, D), lambda i: (i, 0)),
        out_shape=jax.ShapeDtypeStruct((NP, D), jnp.float32),
    )


def kernel(in_feat, edge_index, W_self1, W_neigh1, b1, W_self2, W_neigh2, b2):
    N, D = in_feat.shape
    E = edge_index.shape[1]
    NP = ((N + 1 + NS * K - 1) // (NS * K)) * (NS * K)
    CPT = (E + NW * K - 1) // (NW * K)
    CPT = ((CPT + 7) // 8) * 8
    EP = NW * CPT * K
    pad_e = EP - E

    src = edge_index[0].astype(jnp.int32)
    dst = edge_index[1].astype(jnp.int32)
    pad_ar = jnp.arange(pad_e, dtype=jnp.int32)
    src = jnp.concatenate([src, pad_ar % N])
    dst = jnp.concatenate([dst, N + pad_ar % (NP - N)])
    srcm = src.reshape(NW * CPT, K)
    dstm = dst.reshape(NW * CPT, K)
    dst16 = dst.reshape(EP // L, L)

    x = jnp.concatenate([in_feat, jnp.zeros((NP - N, D), jnp.float32)])

    sc_agg = _build_sc_agg(NP, D, CPT)
    sc_deg = _build_sc_deg(NP, EP // L // NW)
    tc1 = _build_tc_layer(NP, D, relu=True)
    tc2 = _build_tc_layer(NP, D, relu=False)

    aggp1 = sc_agg(x, srcm, dstm)
    degp = sc_deg(dst16)
    h1 = tc1(x, aggp1, degp, W_self1.T, W_neigh1.T, b1.reshape(1, D))
    aggp2 = sc_agg(h1, srcm, dstm)
    out = tc2(h1, aggp2, degp, W_self2.T, W_neigh2.T, b2.reshape(1, D))
    return out[:N]


# speedup vs baseline: 1.1314x; 1.0154x over previous
import jax
import jax.numpy as jnp
from jax import lax
from jax.experimental import pallas as pl
from jax.experimental.pallas import tpu as pltpu
from jax.experimental.pallas import tpu_sc as plsc

NC = 2
NS = 16
NW = NC * NS
L = 16
K = 128


def _sc_mesh():
    return plsc.VectorSubcoreMesh(core_axis_name="c", subcore_axis_name="s",
                                  num_cores=NC, num_subcores=NS)


def _build_sc_agg(NP, D, CPT):
    out_type = jax.ShapeDtypeStruct((NC, NP, D), jnp.float32)
    CG = 16
    G = CPT // CG
    scratch = [
        pltpu.VMEM((CG, K), jnp.int32),
        pltpu.VMEM((CG, K), jnp.int32),
        pltpu.VMEM((CG, K), jnp.int32),
        pltpu.VMEM((CG, K), jnp.int32),
        pltpu.VMEM((K, D), jnp.float32),
        pltpu.VMEM((K, D), jnp.float32),
        pltpu.VMEM_SHARED((NP, D), jnp.float32),
        pltpu.SemaphoreType.DMA,
        pltpu.SemaphoreType.DMA,
        pltpu.SemaphoreType.DMA,
        pltpu.SemaphoreType.DMA,
        pltpu.SemaphoreType.DMA,
    ]

    def body(h_hbm, srcm_hbm, dstm_hbm, agg_out,
             sv0, dv0, sv1, dv1, rows_a, rows_b, agg_sh,
             sem_i, sem_a, sem_b, sem_sa, sem_sb):
        cid = lax.axis_index("c")
        sid = lax.axis_index("s")
        wid = cid * NS + sid
        rpt = NP // NS
        base = sid * rpt
        ibufs = [(sv0, dv0), (sv1, dv1)]

        def stage_idx(g, sv, dv):
            off = wid * CPT + g * CG
            cs = pltpu.async_copy(srcm_hbm.at[pl.ds(off, CG)], sv, sem_i)
            cd = pltpu.async_copy(dstm_hbm.at[pl.ds(off, CG)], dv, sem_i)
            return cs, cd

        cp_s, cp_d = stage_idx(0, sv0, dv0)

        zero16 = jnp.zeros((L,), jnp.float32)

        def zrow(r, _):
            for i in range(D // L):
                rows_a[r, pl.ds(i * L, L)] = zero16
            return 0

        lax.fori_loop(0, K, zrow, 0)
        for t in range(rpt // K):
            pltpu.sync_copy(rows_a, agg_sh.at[pl.ds(base + t * K, K)])

        cp_s.wait()
        cp_d.wait()
        plsc.subcore_barrier()

        def gather(sv, j, buf, sem):
            return pltpu.async_copy(h_hbm.at[sv.at[j]], buf, sem)

        def wait_gather(sv, j, buf, sem):
            pltpu.make_async_copy(h_hbm.at[sv.at[j]], buf, sem).wait()

        def scatter(dv, j, buf):
            pltpu.sync_copy(buf, agg_sh.at[dv.at[j]], add=True)

        gather(sv0, 0, rows_a, sem_a)
        for g in range(G):
            sv, dv = ibufs[g % 2]
            nsv, ndv = ibufs[(g + 1) % 2]
            if g + 1 < G:
                stage_idx(g + 1, nsv, ndv)

            def pair(p, _, sv=sv, dv=dv):
                j0 = 2 * p
                wait_gather(sv, j0, rows_a, sem_a)
                gather(sv, j0 + 1, rows_b, sem_b)
                scatter(dv, j0, rows_a)
                gather(sv, j0 + 2, rows_a, sem_a)
                wait_gather(sv, j0 + 1, rows_b, sem_b)
                scatter(dv, j0 + 1, rows_b)
                return 0

            lax.fori_loop(0, CG // 2 - 1, pair, 0)
            j0 = CG - 2
            wait_gather(sv, j0, rows_a, sem_a)
            gather(sv, j0 + 1, rows_b, sem_b)
            scatter(dv, j0, rows_a)
            if g + 1 < G:
                off = wid * CPT + (g + 1) * CG
                pltpu.make_async_copy(
                    srcm_hbm.at[pl.ds(off, CG)], nsv, sem_i).wait()
                pltpu.make_async_copy(
                    dstm_hbm.at[pl.ds(off, CG)], ndv, sem_i).wait()
                gather(nsv, 0, rows_a, sem_a)
            wait_gather(sv, j0 + 1, rows_b, sem_b)
            scatter(dv, j0 + 1, rows_b)

        plsc.subcore_barrier()
        pltpu.sync_copy(agg_sh.at[pl.ds(base, rpt)],
                        agg_out.at[cid, pl.ds(base, rpt)])

    return pl.kernel(body, out_type=out_type, mesh=_sc_mesh(),
                     scratch_types=scratch)


def _build_sc_deg(NP, EPT16):
    out_type = jax.ShapeDtypeStruct((NW, NP), jnp.float32)
    GG = 128
    scratch = [
        pltpu.VMEM((GG, L), jnp.int32),
        pltpu.VMEM((NP,), jnp.float32),
        pltpu.SemaphoreType.DMA,
    ]

    def body(dst16_hbm, deg_out, dst_v, deg_v, sem):
        cid = lax.axis_index("c")
        sid = lax.axis_index("s")
        wid = cid * NS + sid

        zero16 = jnp.zeros((L,), jnp.float32)

        def zdeg(r, _):
            deg_v[pl.ds(r * L, L)] = zero16
            return 0

        lax.fori_loop(0, NP // L, zdeg, 0)

        def step(q, _):
            for u in range(4):
                d16 = dst_v[4 * q + u, pl.ds(0, L)]
                cnt, last = plsc.scan_count(d16)
                plsc.addupdate_scatter(deg_v, [d16], cnt.astype(jnp.float32),
                                       mask=last)
            return 0

        for g in range(EPT16 // GG):
            pltpu.async_copy(
                dst16_hbm.at[pl.ds(wid * EPT16 + g * GG, GG)], dst_v, sem
            ).wait()
            lax.fori_loop(0, GG // 4, step, 0)

        pltpu.sync_copy(deg_v, deg_out.at[wid])

    return pl.kernel(
        body, out_type=out_type, mesh=_sc_mesh(), scratch_types=scratch,
        compiler_params=pltpu.CompilerParams(needs_layout_passes=False))


def _build_tc_layer(N, NP, D, relu):
    RB = 1024
    grid = (pl.cdiv(N, RB),)

    def body(x_ref, aggp_ref, degp_ref, ws_ref, wn_ref, b_ref, o_ref):
        agg = aggp_ref[0] + aggp_ref[1]
        deg = jnp.sum(jnp.transpose(degp_ref[...]), axis=1, keepdims=True)
        inv = 1.0 / jnp.maximum(deg, 1.0)
        mean = agg * inv
        out = (jnp.dot(x_ref[...], ws_ref[...], preferred_element_type=jnp.float32)
               + jnp.dot(mean, wn_ref[...], preferred_element_type=jnp.float32)
               + b_ref[...])
        if relu:
            out = jnp.maximum(out, 0.0)
        o_ref[...] = out

    return pl.pallas_call(
        body,
        grid=grid,
        in_specs=[
            pl.BlockSpec((RB, D), lambda i: (i, 0)),
            pl.BlockSpec((NC, RB, D), lambda i: (0, i, 0)),
            pl.BlockSpec((NW, RB), lambda i: (0, i)),
            pl.BlockSpec((D, D), lambda i: (0, 0)),
            pl.BlockSpec((D, D), lambda i: (0, 0)),
            pl.BlockSpec((1, D), lambda i: (0, 0)),
        ],
        out_specs=pl.BlockSpec((RB, D), lambda i: (i, 0)),
        out_shape=jax.ShapeDtypeStruct((N, D), jnp.float32),
    )


def kernel(in_feat, edge_index, W_self1, W_neigh1, b1, W_self2, W_neigh2, b2):
    N, D = in_feat.shape
    E = edge_index.shape[1]
    NP = ((N + 1 + NS * K - 1) // (NS * K)) * (NS * K)
    CPT = (E + NW * K - 1) // (NW * K)
    CPT = ((CPT + 7) // 8) * 8
    EP = NW * CPT * K
    pad_e = EP - E

    src = edge_index[0].astype(jnp.int32)
    dst = edge_index[1].astype(jnp.int32)
    pad_ar = jnp.arange(pad_e, dtype=jnp.int32)
    src = jnp.concatenate([src, pad_ar % N])
    dst = jnp.concatenate([dst, N + pad_ar % (NP - N)])
    srcm = src.reshape(NW * CPT, K)
    dstm = dst.reshape(NW * CPT, K)
    dst16 = dst.reshape(EP // L, L)

    sc_agg = _build_sc_agg(NP, D, CPT)
    sc_deg = _build_sc_deg(NP, EP // L // NW)
    tc1 = _build_tc_layer(N, NP, D, relu=True)
    tc2 = _build_tc_layer(N, NP, D, relu=False)

    aggp1 = sc_agg(in_feat, srcm, dstm)
    degp = sc_deg(dst16)
    h1 = tc1(in_feat, aggp1, degp, W_self1.T, W_neigh1.T, b1.reshape(1, D))
    aggp2 = sc_agg(h1, srcm, dstm)
    return tc2(h1, aggp2, degp, W_self2.T, W_neigh2.T, b2.reshape(1, D))
